# baseline (speedup 1.0000x reference)
import functools

import jax
import jax.numpy as jnp
from jax import lax
from jax.experimental import pallas as pl
from jax.experimental.pallas import tpu as pltpu

H = 2048
E = 16
N = 16384
TOPK = 2
BT = 512


def _fused_body(x_ref, w_ref, rwt_ref, set_ref):
    logits = lax.dot_general(
        w_ref[...], x_ref[...],
        dimension_numbers=(((1,), (1,)), ((), ())),
        preferred_element_type=jnp.float32,
    )
    neg_inf = jnp.float32(-jnp.inf)
    xs = [lax.slice(logits, (e, 0), (e + 1, BT)) for e in range(E)]
    m1 = functools.reduce(jnp.maximum, xs)
    idx1 = jnp.zeros((1, BT), jnp.int32)
    for e in reversed(range(E)):
        idx1 = jnp.where(xs[e] == m1, jnp.int32(e), idx1)
    xs2 = [jnp.where(idx1 == jnp.int32(e), neg_inf, xs[e]) for e in range(E)]
    m2 = functools.reduce(jnp.maximum, xs2)
    idx2 = jnp.zeros((1, BT), jnp.int32)
    for e in reversed(range(E)):
        idx2 = jnp.where(xs2[e] == m2, jnp.int32(e), idx2)
    ex = jnp.exp(m2 - m1)
    denom = jnp.float32(1.0) + ex
    rwt_ref[0:1, :] = jnp.float32(1.0) / denom
    rwt_ref[1:2, :] = ex / denom
    set_ref[0:1, :] = idx1
    set_ref[1:2, :] = idx2


def _fused_tc(x, w):
    return pl.pallas_call(
        _fused_body,
        grid=(N // BT,),
        in_specs=[
            pl.BlockSpec((BT, H), lambda i: (i, 0)),
            pl.BlockSpec((E, H), lambda i: (0, 0)),
        ],
        out_specs=[
            pl.BlockSpec((TOPK, BT), lambda i: (0, i)),
            pl.BlockSpec((TOPK, BT), lambda i: (0, i)),
        ],
        out_shape=[
            jax.ShapeDtypeStruct((TOPK, N), jnp.float32),
            jax.ShapeDtypeStruct((TOPK, N), jnp.int32),
        ],
    )(x, w)


def kernel(hidden_states, weight):
    rwt, set_ = _fused_tc(hidden_states, weight)
    routing_weights = jnp.stack([rwt[0], rwt[1]], axis=-1)
    selected_experts = jnp.stack([set_[0], set_[1]], axis=-1)
    return routing_weights, selected_experts


# speedup vs baseline: 1.2886x; 1.2886x over previous
import functools

import jax
import jax.numpy as jnp
from jax import lax
from jax.experimental import pallas as pl
from jax.experimental.pallas import tpu as pltpu

H = 2048
E = 16
N = 16384
TOPK = 2
BT = 1024
GB = BT // 128


def _fused_body(x_ref, w_ref, rw3_ref, se3_ref):
    logits = lax.dot_general(
        w_ref[...], x_ref[...],
        dimension_numbers=(((1,), (1,)), ((), ())),
        preferred_element_type=jnp.float32,
    )
    neg_inf = jnp.float32(-jnp.inf)
    xs = [lax.slice(logits, (e, 0), (e + 1, BT)) for e in range(E)]
    m1 = functools.reduce(jnp.maximum, xs)
    idx1 = jnp.zeros((1, BT), jnp.int32)
    for e in reversed(range(E)):
        idx1 = jnp.where(xs[e] == m1, jnp.int32(e), idx1)
    xs2 = [jnp.where(idx1 == jnp.int32(e), neg_inf, xs[e]) for e in range(E)]
    m2 = functools.reduce(jnp.maximum, xs2)
    idx2 = jnp.zeros((1, BT), jnp.int32)
    for e in reversed(range(E)):
        idx2 = jnp.where(xs2[e] == m2, jnp.int32(e), idx2)
    ex = jnp.exp(m2 - m1)
    denom = jnp.float32(1.0) + ex
    rw3_ref[:, 0, :] = (jnp.float32(1.0) / denom).reshape(GB, 128)
    rw3_ref[:, 1, :] = (ex / denom).reshape(GB, 128)
    se3_ref[:, 0, :] = idx1.reshape(GB, 128)
    se3_ref[:, 1, :] = idx2.reshape(GB, 128)


def _fused_tc(x, w):
    return pl.pallas_call(
        _fused_body,
        grid=(N // BT,),
        in_specs=[
            pl.BlockSpec((BT, H), lambda i: (i, 0)),
            pl.BlockSpec((E, H), lambda i: (0, 0)),
        ],
        out_specs=[
            pl.BlockSpec((GB, TOPK, 128), lambda i: (i, 0, 0)),
            pl.BlockSpec((GB, TOPK, 128), lambda i: (i, 0, 0)),
        ],
        out_shape=[
            jax.ShapeDtypeStruct((N // 128, TOPK, 128), jnp.float32),
            jax.ShapeDtypeStruct((N // 128, TOPK, 128), jnp.int32),
        ],
    )(x, w)


def kernel(hidden_states, weight):
    rw3, se3 = _fused_tc(hidden_states, weight)
    routing_weights = jnp.swapaxes(rw3, 1, 2).reshape(N, TOPK)
    selected_experts = jnp.swapaxes(se3, 1, 2).reshape(N, TOPK)
    return routing_weights, selected_experts
